# baseline (speedup 1.0000x reference)
import jax
import jax.numpy as jnp
from jax.experimental import pallas as pl
from jax.experimental.pallas import tpu as pltpu

_K = 1024
_C = 256
_B = 8
_P = 1024
_BETA = 0.25
_EPS = 1e-12
_ENT_EPS = 1e-4


def _vq_body(z_ref, ew_ref, zq_ref, idx_ref, loss_ref, ent_ref,
             ewn16_ref, ewthi_ref):
    b = pl.program_id(0)

    @pl.when(b == 0)
    def _init():
        ew = ew_ref[...]
        norm = jnp.sqrt(jnp.sum(ew * ew, axis=1, keepdims=True))
        ewn = ew / jnp.maximum(norm, _EPS)
        ewn16_ref[...] = ewn.astype(jnp.bfloat16)
        ewthi_ref[...] = ewn.T.astype(jnp.bfloat16)
        loss_ref[...] = jnp.zeros_like(loss_ref)

    cdims = (((1,), (0,)), ((), ()))
    z = z_ref[0]
    s2 = jnp.sum(z * z, axis=0, keepdims=True)
    n = jnp.maximum(jnp.sqrt(s2), _EPS)
    zn16 = (z / n).astype(jnp.bfloat16)
    CK = 128
    rmax = None
    ridx = None
    for c in range(_K // CK):
        simc = jax.lax.dot_general(
            ewn16_ref[c * CK:(c + 1) * CK, :], zn16, cdims,
            preferred_element_type=jnp.float32)
        cmax = jnp.max(simc, axis=0, keepdims=True)
        kio = jax.lax.broadcasted_iota(jnp.int32, simc.shape, 0)
        cidx = jnp.min(jnp.where(simc == cmax, kio, jnp.int32(2**30)),
                       axis=0, keepdims=True) + (c * CK)
        if rmax is None:
            rmax, ridx = cmax, cidx
        else:
            better = cmax > rmax
            ridx = jnp.where(better, cidx, ridx)
            rmax = jnp.where(better, cmax, rmax)
    idx_ref[pl.ds(b, 1), :] = ridx

    kiota = jax.lax.broadcasted_iota(jnp.int32, (_K, _P), 0)
    oh16 = (kiota == ridx).astype(jnp.bfloat16)
    zq = jax.lax.dot_general(ewthi_ref[...], oh16, cdims,
                             preferred_element_type=jnp.float32)
    zq_ref[0] = zq
    loss_ref[...] += jnp.sum(2.0 - 2.0 * rmax).reshape(1, 1)

    @pl.when(b == _B - 1)
    def _finish():
        loss_ref[...] = ((1.0 + _BETA) / (_B * _P)) * loss_ref[...]
        idxall = idx_ref[...].reshape(1, _B * _P)
        kcol = jax.lax.broadcasted_iota(jnp.int32, (_K, 1), 0)
        cnt = jnp.sum((idxall == kcol).astype(jnp.float32),
                      axis=1, keepdims=True)
        pe = cnt + _ENT_EPS
        probs = pe / jnp.sum(pe)
        ent_ref[...] = -jnp.sum(probs * jnp.log(probs)).reshape(1, 1)


def kernel(z, embedding_weight):
    zr = z.reshape(_B, _C, _P)
    zq, idx, loss, ent = pl.pallas_call(
        _vq_body,
        grid=(_B,),
        in_specs=[
            pl.BlockSpec((1, _C, _P), lambda b: (b, 0, 0)),
            pl.BlockSpec((_K, _C), lambda b: (0, 0)),
        ],
        out_specs=[
            pl.BlockSpec((1, _C, _P), lambda b: (b, 0, 0)),
            pl.BlockSpec((_B, _P), lambda b: (0, 0)),
            pl.BlockSpec((1, 1), lambda b: (0, 0)),
            pl.BlockSpec((1, 1), lambda b: (0, 0)),
        ],
        out_shape=[
            jax.ShapeDtypeStruct((_B, _C, _P), jnp.float32),
            jax.ShapeDtypeStruct((_B, _P), jnp.int32),
            jax.ShapeDtypeStruct((1, 1), jnp.float32),
            jax.ShapeDtypeStruct((1, 1), jnp.float32),
        ],
        scratch_shapes=[
            pltpu.VMEM((_K, _C), jnp.bfloat16),
            pltpu.VMEM((_C, _K), jnp.bfloat16),
        ],
    )(zr, embedding_weight)
    return (zq.reshape(_B, _C, 32, 32), loss[0, 0], ent[0, 0], idx)


# speedup vs baseline: 1.0168x; 1.0168x over previous
import jax
import jax.numpy as jnp
from jax.experimental import pallas as pl
from jax.experimental.pallas import tpu as pltpu

_K = 1024
_C = 256
_B = 8
_P = 1024
_BETA = 0.25
_EPS = 1e-12
_ENT_EPS = 1e-4


def _vq_body(z_ref, ew_ref, zq_ref, idx_ref, loss_ref, ent_ref,
             ewn16_ref, ewthi_ref):
    b = pl.program_id(0)

    @pl.when(b == 0)
    def _init():
        ew = ew_ref[...]
        norm = jnp.sqrt(jnp.sum(ew * ew, axis=1, keepdims=True))
        ewn = ew / jnp.maximum(norm, _EPS)
        ewn16_ref[...] = ewn.astype(jnp.bfloat16)
        ewthi_ref[...] = ewn.T.astype(jnp.bfloat16)
        loss_ref[...] = jnp.zeros_like(loss_ref)

    cdims = (((1,), (0,)), ((), ()))
    z = z_ref[0]
    s2 = jnp.sum(z * z, axis=0, keepdims=True)
    n = jnp.maximum(jnp.sqrt(s2), _EPS)
    zn16 = (z / n).astype(jnp.bfloat16)
    CK = 256
    rmax = None
    ridx = None
    for c in range(_K // CK):
        simc = jax.lax.dot_general(
            ewn16_ref[c * CK:(c + 1) * CK, :], zn16, cdims,
            preferred_element_type=jnp.float32)
        cmax = jnp.max(simc, axis=0, keepdims=True)
        kio = jax.lax.broadcasted_iota(jnp.int32, simc.shape, 0)
        cidx = jnp.min(jnp.where(simc == cmax, kio, jnp.int32(2**30)),
                       axis=0, keepdims=True) + (c * CK)
        if rmax is None:
            rmax, ridx = cmax, cidx
        else:
            better = cmax > rmax
            ridx = jnp.where(better, cidx, ridx)
            rmax = jnp.where(better, cmax, rmax)
    idx_ref[pl.ds(b, 1), :] = ridx

    kiota = jax.lax.broadcasted_iota(jnp.int32, (_K, _P), 0)
    oh16 = (kiota == ridx).astype(jnp.bfloat16)
    zq = jax.lax.dot_general(ewthi_ref[...], oh16, cdims,
                             preferred_element_type=jnp.float32)
    zq_ref[0] = zq
    loss_ref[...] += jnp.sum(2.0 - 2.0 * rmax).reshape(1, 1)

    @pl.when(b == _B - 1)
    def _finish():
        loss_ref[...] = ((1.0 + _BETA) / (_B * _P)) * loss_ref[...]
        idxall = idx_ref[...].reshape(1, _B * _P)
        kcol = jax.lax.broadcasted_iota(jnp.int32, (_K, 1), 0)
        cnt = jnp.sum((idxall == kcol).astype(jnp.float32),
                      axis=1, keepdims=True)
        pe = cnt + _ENT_EPS
        probs = pe / jnp.sum(pe)
        ent_ref[...] = -jnp.sum(probs * jnp.log(probs)).reshape(1, 1)


def kernel(z, embedding_weight):
    zr = z.reshape(_B, _C, _P)
    zq, idx, loss, ent = pl.pallas_call(
        _vq_body,
        grid=(_B,),
        in_specs=[
            pl.BlockSpec((1, _C, _P), lambda b: (b, 0, 0)),
            pl.BlockSpec((_K, _C), lambda b: (0, 0)),
        ],
        out_specs=[
            pl.BlockSpec((1, _C, _P), lambda b: (b, 0, 0)),
            pl.BlockSpec((_B, _P), lambda b: (0, 0)),
            pl.BlockSpec((1, 1), lambda b: (0, 0)),
            pl.BlockSpec((1, 1), lambda b: (0, 0)),
        ],
        out_shape=[
            jax.ShapeDtypeStruct((_B, _C, _P), jnp.float32),
            jax.ShapeDtypeStruct((_B, _P), jnp.int32),
            jax.ShapeDtypeStruct((1, 1), jnp.float32),
            jax.ShapeDtypeStruct((1, 1), jnp.float32),
        ],
        scratch_shapes=[
            pltpu.VMEM((_K, _C), jnp.bfloat16),
            pltpu.VMEM((_C, _K), jnp.bfloat16),
        ],
    )(zr, embedding_weight)
    return (zq.reshape(_B, _C, 32, 32), loss[0, 0], ent[0, 0], idx)
